# baseline (speedup 1.0000x reference)
import functools

import jax
import jax.numpy as jnp
from jax import lax
from jax.experimental import pallas as pl
from jax.experimental.pallas import tpu as pltpu
from jax.experimental.pallas import tpu_sc as plsc

_VOCAB = 100000
_DIM = 768
_B = 4
_S = 2048
_N_IDX = _B * _S
_NC = 2
_NS = 16
_NW = _NC * _NS
_PER_W = _N_IDX // _NW
_CHUNK = 32
_NCHUNK = _PER_W // _CHUNK
_NSTEPS = 3 * _NCHUNK
_NBUF = 3
_GROUPS = _DIM // 32


def _make_kernel():
  mesh = plsc.VectorSubcoreMesh(
      core_axis_name="c", subcore_axis_name="s",
      num_cores=_NC, num_subcores=_NS)

  @functools.partial(
      pl.kernel,
      out_type=[jax.ShapeDtypeStruct((_N_IDX, _DIM), jnp.bfloat16)] * 6,
      mesh=mesh,
      compiler_params=pltpu.CompilerParams(needs_layout_passes=False),
      scratch_types=[
          pltpu.VMEM((_PER_W,), jnp.int32),
          pltpu.VMEM((_CHUNK, _DIM), jnp.float32),
          pltpu.VMEM((_CHUNK, _DIM), jnp.float32),
          pltpu.VMEM((_CHUNK, _DIM), jnp.float32),
          pltpu.VMEM((_CHUNK, _DIM), jnp.bfloat16),
          pltpu.VMEM((_CHUNK, _DIM), jnp.bfloat16),
          pltpu.VMEM((_CHUNK, _DIM), jnp.bfloat16),
          pltpu.SemaphoreType.DMA,
          pltpu.SemaphoreType.DMA,
          pltpu.SemaphoreType.DMA,
          pltpu.SemaphoreType.DMA,
          pltpu.SemaphoreType.DMA,
          pltpu.SemaphoreType.DMA,
          pltpu.SemaphoreType.DMA,
      ],
  )
  def emb_kernel(idx_hbm, w0, w1, w2, o0, o1, o2, o3, o4, o5,
                 idx_v, in0, in1, in2, ob0, ob1, ob2,
                 gs0, gs1, gs2, ss0, ss1, ss2, dup_sem):
    wid = lax.axis_index("s") * _NC + lax.axis_index("c")
    base = wid * _PER_W
    pltpu.sync_copy(idx_hbm.at[base // _S, pl.ds(base % _S, _PER_W)], idx_v)

    tables = (w0, w1, w2)
    outs = ((o0, o3), (o1, o4), (o2, o5))
    inbufs = (in0, in1, in2)
    outbufs = (ob0, ob1, ob2)
    gsems = (gs0, gs1, gs2)
    ssems = (ss0, ss1, ss2)

    two_iota = lax.iota(jnp.int32, 16) * 2

    def start_gather(step, p):
      for t in range(3):
        @pl.when(jnp.logical_and(step >= t * _NCHUNK,
                                 step < (t + 1) * _NCHUNK))
        def _():
          c = step - t * _NCHUNK
          pltpu.async_copy(
              tables[t].at[idx_v.at[pl.ds(c * _CHUNK, _CHUNK)]],
              inbufs[p], gsems[p])

    def start_stores(step, p):
      for t in range(3):
        @pl.when(jnp.logical_and(step >= t * _NCHUNK,
                                 step < (t + 1) * _NCHUNK))
        def _():
          dst = pl.ds(base + (step - t * _NCHUNK) * _CHUNK, _CHUNK)
          pltpu.async_copy(outbufs[p], outs[t][0].at[dst], ssems[p])

    def start_dup(step):
      for t in range(3):
        @pl.when(jnp.logical_and(step >= t * _NCHUNK,
                                 step < (t + 1) * _NCHUNK))
        def _():
          dst = pl.ds(base + (step - t * _NCHUNK) * _CHUNK, _CHUNK)
          pltpu.async_copy(outs[t][0].at[dst], outs[t][1].at[dst], dup_sem)

    def wait_gather(p):
      pltpu.make_async_copy(
          tables[0].at[pl.ds(0, _CHUNK)], inbufs[p], gsems[p]).wait()

    def wait_stores(p):
      pltpu.make_async_copy(
          outbufs[p], outs[0][0].at[pl.ds(0, _CHUNK)], ssems[p]).wait()

    def convert(inb, outb):
      @plsc.parallel_loop(0, _CHUNK)
      def _rows(r):
        rr = jnp.full((16,), r, jnp.int32)

        @plsc.parallel_loop(0, _GROUPS, unroll=4)
        def _groups(k):
          col_e = k * 32 + two_iota
          a = plsc.load_gather(inb, [rr, col_e])
          b = plsc.load_gather(inb, [rr, col_e + 1])
          outb[r, pl.ds(k * 32, 32)] = plsc.pack(
              a, b, format=plsc.PackFormat.INTERLEAVED)

    start_gather(jnp.int32(0), 0)
    start_gather(jnp.int32(1), 1)

    @pl.loop(0, _NSTEPS // _NBUF)
    def _steps(i):
      for par in range(_NBUF):
        step = i * _NBUF + par
        wait_gather(par)

        @pl.when(step + 2 < _NSTEPS)
        def _():
          start_gather(step + 2, (par + 2) % _NBUF)

        @pl.when(i >= 1)
        def _():
          wait_stores(par)
          start_dup(step - _NBUF)

        convert(inbufs[par], outbufs[par])
        start_stores(step, par)

    for par in range(_NBUF):
      wait_stores(par)
      start_dup(jnp.int32(_NSTEPS - _NBUF + par))
    for _ in range(_NSTEPS):
      pltpu.make_async_copy(
          outs[0][0].at[pl.ds(0, _CHUNK)], outs[0][1].at[pl.ds(0, _CHUNK)],
          dup_sem).wait()

  return emb_kernel


_emb_kernel = _make_kernel()


def kernel(inputs, W0, W1, W2):
  e0, e1, e2, e3, e4, e5 = _emb_kernel(inputs, W0, W1, W2)
  shape = (_B, _S, _DIM)
  return (e0.reshape(shape), e1.reshape(shape), e2.reshape(shape),
          e3.reshape(shape), e4.reshape(shape), e5.reshape(shape))


# speedup vs baseline: 15.8469x; 15.8469x over previous
import functools

import jax
import jax.numpy as jnp
from jax import lax
from jax.experimental import pallas as pl
from jax.experimental.pallas import tpu as pltpu
from jax.experimental.pallas import tpu_sc as plsc

_VOCAB = 100000
_DIM = 768
_B = 4
_S = 2048
_N_IDX = _B * _S
_NC = 2
_NS = 16
_NW = _NC * _NS
_PER_W = _N_IDX // _NW
_CHUNK = 16
_NCHUNK = _PER_W // _CHUNK
_NSTEPS = 3 * _NCHUNK
_NBUF = 4
_AHEAD = _NBUF - 1
_GROUPS = _DIM // 32


def _make_kernel():
  mesh = plsc.VectorSubcoreMesh(
      core_axis_name="c", subcore_axis_name="s",
      num_cores=_NC, num_subcores=_NS)

  @functools.partial(
      pl.kernel,
      out_type=[jax.ShapeDtypeStruct((_N_IDX, _DIM), jnp.bfloat16)] * 6,
      mesh=mesh,
      compiler_params=pltpu.CompilerParams(needs_layout_passes=False),
      scratch_types=[
          pltpu.VMEM((_PER_W,), jnp.int32),
          pltpu.VMEM((_CHUNK, _DIM), jnp.float32),
          pltpu.VMEM((_CHUNK, _DIM), jnp.float32),
          pltpu.VMEM((_CHUNK, _DIM), jnp.float32),
          pltpu.VMEM((_CHUNK, _DIM), jnp.float32),
          pltpu.VMEM((_CHUNK, _DIM), jnp.bfloat16),
          pltpu.VMEM((_CHUNK, _DIM), jnp.bfloat16),
          pltpu.VMEM((_CHUNK, _DIM), jnp.bfloat16),
          pltpu.VMEM((_CHUNK, _DIM), jnp.bfloat16),
          pltpu.SemaphoreType.DMA,
          pltpu.SemaphoreType.DMA,
          pltpu.SemaphoreType.DMA,
          pltpu.SemaphoreType.DMA,
          pltpu.SemaphoreType.DMA,
          pltpu.SemaphoreType.DMA,
          pltpu.SemaphoreType.DMA,
          pltpu.SemaphoreType.DMA,
      ],
  )
  def emb_kernel(idx_hbm, w0, w1, w2, o0, o1, o2, o3, o4, o5,
                 idx_v, in0, in1, in2, in3, ob0, ob1, ob2, ob3,
                 gs0, gs1, gs2, gs3, ss0, ss1, ss2, ss3):
    wid = lax.axis_index("s") * _NC + lax.axis_index("c")
    base = wid * _PER_W
    pltpu.sync_copy(idx_hbm.at[base // _S, pl.ds(base % _S, _PER_W)], idx_v)

    tables = (w0, w1, w2)
    outs = ((o0, o3), (o1, o4), (o2, o5))
    inbufs = (in0, in1, in2, in3)
    outbufs = (ob0, ob1, ob2, ob3)
    gsems = (gs0, gs1, gs2, gs3)
    ssems = (ss0, ss1, ss2, ss3)

    two_iota = lax.iota(jnp.int32, 16) * 2

    def start_gather(step, p):
      for t in range(3):
        @pl.when(jnp.logical_and(step >= t * _NCHUNK,
                                 step < (t + 1) * _NCHUNK))
        def _():
          c = step - t * _NCHUNK
          pltpu.async_copy(
              tables[t].at[idx_v.at[pl.ds(c * _CHUNK, _CHUNK)]],
              inbufs[p], gsems[p])

    def start_stores(step, p):
      for t in range(3):
        @pl.when(jnp.logical_and(step >= t * _NCHUNK,
                                 step < (t + 1) * _NCHUNK))
        def _():
          dst = pl.ds(base + (step - t * _NCHUNK) * _CHUNK, _CHUNK)
          for o in outs[t]:
            pltpu.async_copy(outbufs[p], o.at[dst], ssems[p])

    def wait_gather(p):
      pltpu.make_async_copy(
          tables[0].at[pl.ds(0, _CHUNK)], inbufs[p], gsems[p]).wait()

    def wait_stores(p):
      for _ in range(2):
        pltpu.make_async_copy(
            outbufs[p], outs[0][0].at[pl.ds(0, _CHUNK)], ssems[p]).wait()

    def convert(inb, outb):
      @plsc.parallel_loop(0, _CHUNK)
      def _rows(r):
        rr = jnp.full((16,), r, jnp.int32)

        @plsc.parallel_loop(0, _GROUPS, unroll=4)
        def _groups(k):
          col_e = k * 32 + two_iota
          a = plsc.load_gather(inb, [rr, col_e])
          b = plsc.load_gather(inb, [rr, col_e + 1])
          outb[r, pl.ds(k * 32, 32)] = plsc.pack(
              a, b, format=plsc.PackFormat.INTERLEAVED)

    for s in range(_AHEAD):
      start_gather(jnp.int32(s), s)

    @pl.loop(0, _NSTEPS // _NBUF)
    def _steps(i):
      for par in range(_NBUF):
        step = i * _NBUF + par
        wait_gather(par)

        @pl.when(step + _AHEAD < _NSTEPS)
        def _():
          start_gather(step + _AHEAD, (par + _AHEAD) % _NBUF)

        @pl.when(i >= 1)
        def _():
          wait_stores(par)

        convert(inbufs[par], outbufs[par])
        start_stores(step, par)

    for p in range(_NBUF):
      wait_stores(p)

  return emb_kernel


_emb_kernel = _make_kernel()


def kernel(inputs, W0, W1, W2):
  e0, e1, e2, e3, e4, e5 = _emb_kernel(inputs, W0, W1, W2)
  shape = (_B, _S, _DIM)
  return (e0.reshape(shape), e1.reshape(shape), e2.reshape(shape),
          e3.reshape(shape), e4.reshape(shape), e5.reshape(shape))
